# Initial kernel scaffold; baseline (speedup 1.0000x reference)
import jax
import jax.numpy as jnp
from jax.experimental import pallas as pl


def kernel(feat, spat, feat_edge_index, spat_edge_index, fc_W, fc_b, cnn_fc_W, cnn_fc_b, fconv1_W, fconv1_b, fconv2_W, fconv2_b, sconv1_W, sconv1_b, sconv2_W, sconv2_b, proj1_W, proj1_b, proj2_W, proj2_b):
    raise NotImplementedError("write your pallas kernel here")


# speedup vs baseline: 21.4046x; 21.4046x over previous
import functools

import jax
import jax.numpy as jnp
from jax import lax
from jax.experimental import pallas as pl
from jax.experimental.pallas import tpu as pltpu
from jax.experimental.pallas import tpu_sc as plsc

NC = 2
NS = 16
NTILES = NC * NS
CH = 128


def _fill_1d(ref, m, value):
    def body(i, carry):
        ref[pl.ds(i * 16, 16)] = jnp.full((16,), value, jnp.float32)
        return carry
    lax.fori_loop(0, m // 16, body, 0)


def _fill_2d(ref, m, w):
    def body(i, carry):
        for c in range(w // 16):
            ref[i, pl.ds(c * 16, 16)] = jnp.zeros((16,), jnp.float32)
        return carry
    lax.fori_loop(0, m, body, 0)


def _make_sc_deg(n1, cpt):
    rpt = n1 // NS

    @functools.partial(
        pl.kernel,
        out_type=[jax.ShapeDtypeStruct((n1,), jnp.float32)] * 4,
        mesh=plsc.VectorSubcoreMesh(core_axis_name="c", subcore_axis_name="s"),
        compiler_params=pltpu.CompilerParams(use_tc_tiling_on_sc=False),
        scratch_types=[
            pltpu.VMEM((cpt, CH), jnp.int32),
            pltpu.VMEM((CH,), jnp.float32),
            pltpu.VMEM((rpt,), jnp.float32),
            pltpu.VMEM_SHARED((n1,), jnp.float32),
            pltpu.VMEM_SHARED((n1,), jnp.float32),
        ],
    )
    def deg_kernel(colf_hbm, cols_hbm,
                   outf0_hbm, outf1_hbm, outs0_hbm, outs1_hbm,
                   colbuf, onesb, zbuf, accf_sh, accs_sh):
        cid = lax.axis_index("c")
        sid = lax.axis_index("s")
        rs = pl.ds(sid * rpt, rpt)
        _fill_1d(onesb, CH, 1.0)
        _fill_1d(zbuf, rpt, 0.0)
        pltpu.sync_copy(zbuf, accf_sh.at[rs])
        pltpu.sync_copy(zbuf, accs_sh.at[rs])
        plsc.subcore_barrier()
        base = (cid * NS + sid) * cpt
        for col_hbm, acc in ((colf_hbm, accf_sh), (cols_hbm, accs_sh)):
            pltpu.sync_copy(col_hbm.at[pl.ds(base, cpt), :], colbuf)

            def chunk(j, carry):
                pltpu.sync_copy(onesb, acc.at[colbuf.at[j]], add=True)
                return carry

            lax.fori_loop(0, cpt, chunk, 0)
        plsc.subcore_barrier()

        @pl.when(cid == 0)
        def _():
            pltpu.sync_copy(accf_sh.at[rs], outf0_hbm.at[rs])
            pltpu.sync_copy(accs_sh.at[rs], outs0_hbm.at[rs])

        @pl.when(cid == 1)
        def _():
            pltpu.sync_copy(accf_sh.at[rs], outf1_hbm.at[rs])
            pltpu.sync_copy(accs_sh.at[rs], outs1_hbm.at[rs])

    return deg_kernel


def _make_sc_scatter(n1, cpt, wf, ws):
    rpt = n1 // NS

    @functools.partial(
        pl.kernel,
        out_type=[
            jax.ShapeDtypeStruct((NC, n1, wf), jnp.float32),
            jax.ShapeDtypeStruct((NC, n1, ws), jnp.float32),
        ],
        mesh=plsc.VectorSubcoreMesh(core_axis_name="c", subcore_axis_name="s"),
        compiler_params=pltpu.CompilerParams(use_tc_tiling_on_sc=False),
        scratch_types=[
            pltpu.VMEM((cpt, CH), jnp.int32),
            pltpu.VMEM((cpt, CH), jnp.int32),
            pltpu.VMEM((CH, wf), jnp.float32),
            pltpu.VMEM((CH, ws), jnp.float32),
            pltpu.VMEM((rpt, wf), jnp.float32),
            pltpu.VMEM((rpt, ws), jnp.float32),
            pltpu.VMEM_SHARED((n1, wf), jnp.float32),
            pltpu.VMEM_SHARED((n1, ws), jnp.float32),
        ],
    )
    def scatter_kernel(tabf_hbm, tabs_hbm, rowf_hbm, colf_hbm, rows_hbm,
                       cols_hbm, outf_hbm, outs_hbm,
                       rowbuf, colbuf, gbuf_f, gbuf_s, zbuf_f, zbuf_s,
                       accf_sh, accs_sh):
        cid = lax.axis_index("c")
        sid = lax.axis_index("s")
        rs = pl.ds(sid * rpt, rpt)
        _fill_2d(zbuf_f, rpt, wf)
        _fill_2d(zbuf_s, rpt, ws)
        pltpu.sync_copy(zbuf_f, accf_sh.at[rs])
        pltpu.sync_copy(zbuf_s, accs_sh.at[rs])
        plsc.subcore_barrier()
        base = (cid * NS + sid) * cpt
        for tab, row_hbm, col_hbm, gbuf, acc in (
                (tabf_hbm, rowf_hbm, colf_hbm, gbuf_f, accf_sh),
                (tabs_hbm, rows_hbm, cols_hbm, gbuf_s, accs_sh)):
            pltpu.sync_copy(row_hbm.at[pl.ds(base, cpt), :], rowbuf)
            pltpu.sync_copy(col_hbm.at[pl.ds(base, cpt), :], colbuf)

            def chunk(j, carry):
                pltpu.sync_copy(tab.at[rowbuf.at[j]], gbuf)
                pltpu.sync_copy(gbuf, acc.at[colbuf.at[j]], add=True)
                return carry

            lax.fori_loop(0, cpt, chunk, 0)
        plsc.subcore_barrier()
        pltpu.sync_copy(accf_sh.at[rs], outf_hbm.at[cid, rs, :])
        pltpu.sync_copy(accs_sh.at[rs], outs_hbm.at[cid, rs, :])

    return scatter_kernel


def _dinv(d0, d1):
    return lax.rsqrt(d0[...] + d1[...] + 1.0)


def _tc_a_body(feat, spat, df0, df1, ds0, ds1, fcW, fcb, cW, cb, W1f, W1s,
               h1f_o, h1s_o):
    dinvf = _dinv(df0, df1)
    dinvs = _dinv(ds0, ds1)
    x0f = jnp.maximum(feat[...] @ fcW[...] + fcb[...], 0.0)
    h1f_o[...] = (x0f @ W1f[...]) * dinvf
    x0s = jnp.maximum(spat[...] @ cW[...] + cb[...], 0.0)
    h1s_o[...] = (x0s @ W1s[...]) * dinvs


def _tc_b_body(pf, ps, h1f, h1s, df0, df1, ds0, ds1, b1f, b1s, W2s,
               yf_o, h2s_o):
    dinvf = _dinv(df0, df1)
    dinvs = _dinv(ds0, ds1)
    x1f = jnp.maximum(dinvf * (pf[0] + pf[1] + h1f[...]) + b1f[...], 0.0)
    yf_o[...] = dinvf * x1f
    x1s = jnp.maximum(dinvs * (ps[0] + ps[1] + h1s[...]) + b1s[...], 0.0)
    h2 = (x1s @ W2s[...]) * dinvs
    h2s_o[...] = jnp.concatenate(
        [h2, jnp.zeros((h2.shape[0], 5), jnp.float32)], axis=1)


def _tc_c_body(qf, qs, yf, h2s, df0, df1, ds0, ds1, W2f, b2f, b2s,
               p1W, p1b, p2W, p2b, out_o):
    dinvf = _dinv(df0, df1)
    dinvs = _dinv(ds0, ds1)
    xf2 = (dinvf * (qf[0] + qf[1] + yf[...])) @ W2f[...] + b2f[...]
    xs2 = dinvs * (qs[0] + qs[1] + h2s[...])[:, :11] + b2s[...]
    x = jnp.maximum(jnp.concatenate([xf2, xs2], axis=1), 0.0)
    x = jnp.maximum(x @ p1W[...] + p1b[...], 0.0)
    out_o[...] = x @ p2W[...] + p2b[...]


def _full(shape):
    return pl.BlockSpec(shape, lambda i: tuple(0 for _ in shape))


def _rows(r, *rest):
    ndims = 1 + len(rest)
    if ndims == 1:
        return pl.BlockSpec((r,), lambda i: (i,))
    return pl.BlockSpec((r,) + tuple(rest), lambda i: (i,) + (0,) * len(rest))


def _part(lead, r, *rest):
    return pl.BlockSpec((lead, r) + tuple(rest),
                        lambda i: (0, i) + (0,) * len(rest))


def kernel(feat, spat, feat_edge_index, spat_edge_index,
           fc_W, fc_b, cnn_fc_W, cnn_fc_b,
           fconv1_W, fconv1_b, fconv2_W, fconv2_b,
           sconv1_W, sconv1_b, sconv2_W, sconv2_b,
           proj1_W, proj1_b, proj2_W, proj2_b):
    n, d_in = feat.shape
    e = feat_edge_index.shape[1]
    n1 = ((n + 1 + NS * 16 - 1) // (NS * 16)) * (NS * 16)
    epad = ((e + NTILES * CH * 8 - 1) // (NTILES * CH * 8)) * (NTILES * CH * 8)
    cpt = epad // (NTILES * CH)

    def prep(ei):
        row = jnp.concatenate(
            [ei[0], jnp.zeros((epad - e,), jnp.int32)]).reshape(epad // CH, CH)
        col = jnp.concatenate(
            [ei[1], jnp.full((epad - e,), n, jnp.int32)]).reshape(epad // CH, CH)
        return row, col

    rowf, colf = prep(feat_edge_index)
    rows_, cols_ = prep(spat_edge_index)

    df0, df1, ds0, ds1 = _make_sc_deg(n1, cpt)(colf, cols_)
    df0 = df0.reshape(n1, 1)
    df1 = df1.reshape(n1, 1)
    ds0 = ds0.reshape(n1, 1)
    ds1 = ds1.reshape(n1, 1)

    r = 1000
    grid = (n // r,)
    w_in = d_in

    h1f, h1s = pl.pallas_call(
        _tc_a_body,
        grid=grid,
        in_specs=[
            _rows(r, w_in), _rows(r, w_in),
            _rows(r, 1), _rows(r, 1), _rows(r, 1), _rows(r, 1),
            _full((w_in, 32)), _full((1, 32)),
            _full((w_in, 32)), _full((1, 32)),
            _full((32, 32)), _full((32, 32)),
        ],
        out_specs=[_rows(r, 32), _rows(r, 32)],
        out_shape=[jax.ShapeDtypeStruct((n, 32), jnp.float32)] * 2,
    )(feat, spat, df0, df1, ds0, ds1,
      fc_W, fc_b.reshape(1, 32), cnn_fc_W, cnn_fc_b.reshape(1, 32),
      fconv1_W, sconv1_W)

    pf, ps = _make_sc_scatter(n1, cpt, 32, 32)(
        h1f, h1s, rowf, colf, rows_, cols_)

    yf, h2s = pl.pallas_call(
        _tc_b_body,
        grid=grid,
        in_specs=[
            _part(NC, r, 32), _part(NC, r, 32),
            _rows(r, 32), _rows(r, 32),
            _rows(r, 1), _rows(r, 1), _rows(r, 1), _rows(r, 1),
            _full((1, 32)), _full((1, 32)), _full((32, 11)),
        ],
        out_specs=[_rows(r, 32), _rows(r, 16)],
        out_shape=[jax.ShapeDtypeStruct((n, 32), jnp.float32),
                   jax.ShapeDtypeStruct((n, 16), jnp.float32)],
    )(pf, ps, h1f, h1s, df0, df1, ds0, ds1,
      fconv1_b.reshape(1, 32), sconv1_b.reshape(1, 32), sconv2_W)

    qf, qs = _make_sc_scatter(n1, cpt, 32, 16)(
        yf, h2s, rowf, colf, rows_, cols_)

    out = pl.pallas_call(
        _tc_c_body,
        grid=grid,
        in_specs=[
            _part(NC, r, 32), _part(NC, r, 16),
            _rows(r, 32), _rows(r, 16),
            _rows(r, 1), _rows(r, 1), _rows(r, 1), _rows(r, 1),
            _full((32, 33)), _full((1, 33)), _full((1, 11)),
            _full((44, 33)), _full((1, 33)), _full((33, 128)), _full((1, 128)),
        ],
        out_specs=_rows(r, 128),
        out_shape=jax.ShapeDtypeStruct((n, 128), jnp.float32),
    )(qf, qs, yf, h2s, df0, df1, ds0, ds1,
      fconv2_W, fconv2_b.reshape(1, 33), sconv2_b.reshape(1, 11),
      proj1_W, proj1_b.reshape(1, 33), proj2_W, proj2_b.reshape(1, 128))

    return out
